# baseline (speedup 1.0000x reference)
import functools

import jax
import jax.numpy as jnp
from jax import lax
from jax.experimental import pallas as pl
from jax.experimental.pallas import tpu as pltpu
from jax.experimental.pallas import tpu_sc as plsc

DIM = 768
HID = 4 * DIM
GDIM = 2 * DIM
E = 8
K = 2
N = 2048
NE = N * K
BLK = 256
NBLK = NE // BLK + E
NSLOT = NBLK * BLK
L = 16
HALF = N // 2

_NEG = -3.0e38


def _silu(v):
    return v * lax.logistic(v)


def _gate_body(x_ref, gw1_ref, gb1_ref, gw2_ref, gb2_ref,
               i1_ref, i2_ref, wa_ref, wb_ref):
    xt = x_ref[...]
    hg = _silu(jnp.dot(xt, gw1_ref[...],
                       preferred_element_type=jnp.float32) + gb1_ref[...])
    g = jnp.dot(hg, gw2_ref[...],
                preferred_element_type=jnp.float32) + gb2_ref[...]
    e_idx = lax.broadcasted_iota(jnp.int32, (N, E), 1)
    max1 = jnp.max(g, axis=1, keepdims=True)
    i1 = jnp.min(jnp.where(g == max1, e_idx, E), axis=1, keepdims=True)
    g2 = jnp.where(e_idx == i1, _NEG, g)
    max2 = jnp.max(g2, axis=1, keepdims=True)
    i2 = jnp.min(jnp.where(g2 == max2, e_idx, E), axis=1, keepdims=True)
    p = jnp.exp(max2 - max1)
    wa = 1.0 / (1.0 + p)
    i1_ref[...] = jnp.min(jnp.where(g == max1, e_idx, E), axis=1)
    i2_ref[...] = jnp.min(jnp.where(g2 == max2, e_idx, E), axis=1)
    wa_ref[...] = wa[:, 0]
    wb_ref[...] = 1.0 - wa[:, 0]


def _gate_call(x, gw1, gb1, gw2, gb2):
    return pl.pallas_call(
        _gate_body,
        out_shape=(
            jax.ShapeDtypeStruct((N,), jnp.int32),
            jax.ShapeDtypeStruct((N,), jnp.int32),
            jax.ShapeDtypeStruct((N,), jnp.float32),
            jax.ShapeDtypeStruct((N,), jnp.float32),
        ),
    )(x, gw1, gb1, gw2, gb2)


def _route_body(i1_hbm, i2_hbm, wa_hbm, wb_hbm, x_hbm,
                srctok_hbm, srcw_hbm, blkm_hbm, xs_hbm,
                eids_v, ws_v, tokb_v, wbuf_v, rows_v, idx64_v, blk_v, sem):
    c = lax.axis_index("c")
    s = lax.axis_index("s")
    e_me = s * 2 + c

    @pl.when(s < 4)
    def _active():
        lane = lax.iota(jnp.int32, 16)
        pltpu.sync_copy(i1_hbm, eids_v.at[pl.ds(0, N)])
        pltpu.sync_copy(i2_hbm, eids_v.at[pl.ds(N, N)])
        pltpu.sync_copy(wa_hbm, ws_v.at[pl.ds(0, N)])
        pltpu.sync_copy(wb_hbm, ws_v.at[pl.ds(N, N)])

        def cbody(i, cnts):
            v = eids_v[pl.ds(i * 16, 16)]
            return tuple(
                cnts[e2] + jnp.sum(jnp.where(v == e2, 1, 0))
                for e2 in range(E))
        counts = lax.fori_loop(
            0, NE // 16, cbody,
            tuple(jnp.int32(0) for _ in range(E)))

        nblk = [lax.shift_right_logical(counts[e2] + (BLK - 1), 8)
                for e2 in range(E)]
        base_s = jnp.int32(0)
        nblk_s = jnp.int32(0)
        for e2 in range(E):
            base_s = base_s + jnp.where(e2 < e_me, nblk[e2] * BLK, 0)
            nblk_s = nblk_s + jnp.where(e2 == e_me, nblk[e2], 0)

        zero_i = jnp.zeros((16,), jnp.int32)

        def zbody(i, _):
            tokb_v[pl.ds(i * 16, 16)] = zero_i
            wbuf_v[pl.ds(i * 16, 16)] = jnp.zeros((16,), jnp.float32)
            return 0
        lax.fori_loop(0, NSLOT // 16, zbody, 0)

        def pbody(i, off):
            v = eids_v[pl.ds(i * 16, 16)]
            m = v == e_me
            p = i * 16 + lane
            tok = jnp.bitwise_and(p, N - 1)
            wv = ws_v[pl.ds(i * 16, 16)]
            cm = plsc.cumsum(jnp.where(m, 1, 0))
            slot = base_s + off + cm - 1
            plsc.store_scatter(tokb_v, [slot], tok, mask=m)
            plsc.store_scatter(wbuf_v, [slot], wv, mask=m)
            return off + jnp.max(cm)
        lax.fori_loop(0, NE // 16, pbody, jnp.int32(0))

        def dbody(j, _):
            sb = pl.multiple_of(base_s + j * BLK, BLK)
            pltpu.sync_copy(tokb_v.at[pl.ds(sb, BLK)],
                            srctok_hbm.at[pl.ds(sb, BLK)])
            pltpu.sync_copy(wbuf_v.at[pl.ds(sb, BLK)],
                            srcw_hbm.at[pl.ds(sb, BLK)])
            for sub in range(BLK // 64):
                so = pl.multiple_of(sb + sub * 64, 64)
                pltpu.async_copy(x_hbm.at[tokb_v.at[pl.ds(so, 64)]],
                                 rows_v, sem).wait()
                pltpu.sync_copy(rows_v, xs_hbm.at[pl.ds(so, 64)])
            return 0
        lax.fori_loop(0, nblk_s, dbody, 0)

        @pl.when((c == 0) & (s == 0))
        def _meta():
            cum = jnp.int32(0)
            cums = []
            for e2 in range(E):
                cum = cum + nblk[e2]
                cums.append(cum)
            for step in range(2):
                b = step * 16 + lane
                eid = jnp.zeros((16,), jnp.int32)
                for e2 in range(E):
                    eid = eid + jnp.where(b >= cums[e2], 1, 0)
                blk_v[pl.ds(step * 16, 16)] = eid
            pltpu.sync_copy(blk_v, blkm_hbm)


def _route_call(i1, i2, wa, wb, x):
    mesh = plsc.VectorSubcoreMesh(core_axis_name="c", subcore_axis_name="s",
                                  num_cores=2, num_subcores=16)
    f = pl.kernel(
        _route_body,
        out_type=(
            jax.ShapeDtypeStruct((NSLOT,), jnp.int32),
            jax.ShapeDtypeStruct((NSLOT,), jnp.float32),
            jax.ShapeDtypeStruct((32,), jnp.int32),
            jax.ShapeDtypeStruct((NSLOT, DIM), jnp.float32),
        ),
        mesh=mesh,
        scratch_types=[
            pltpu.VMEM((NE,), jnp.int32),
            pltpu.VMEM((NE,), jnp.float32),
            pltpu.VMEM((NSLOT,), jnp.int32),
            pltpu.VMEM((NSLOT,), jnp.float32),
            pltpu.VMEM((64, DIM), jnp.float32),
            pltpu.VMEM((64,), jnp.int32),
            pltpu.VMEM((32,), jnp.int32),
            pltpu.SemaphoreType.DMA,
        ],
        compiler_params=pltpu.CompilerParams(needs_layout_passes=False),
    )
    return f(i1, i2, wa, wb, x)


def _group_body(meta_ref, xs_ref, w1_ref, b1_ref, w2_ref, b2_ref, tok_ref,
                w_ref, out_ref):
    b = pl.program_id(0)
    e = meta_ref[b]

    @pl.when(b == 0)
    def _init():
        out_ref[...] = jnp.zeros_like(out_ref)

    @pl.when(e < E)
    def _compute():
        xt = xs_ref[...]
        h = _silu(jnp.dot(xt, w1_ref[...],
                          preferred_element_type=jnp.float32) + b1_ref[...])
        y0 = jnp.dot(h, w2_ref[0], preferred_element_type=jnp.float32)
        onehot = (lax.broadcasted_iota(jnp.int32, (1, E), 1) == e
                  ).astype(jnp.float32)
        y0 = y0 + jnp.dot(onehot, b2_ref[...],
                          preferred_element_type=jnp.float32)
        tok_row = tok_ref[...].reshape(1, BLK)
        w_row = w_ref[...].reshape(1, BLK)
        n_i = lax.broadcasted_iota(jnp.int32, (N, BLK), 0)
        scat = jnp.where(n_i == tok_row, jnp.broadcast_to(w_row, (N, BLK)),
                         0.0)
        out_ref[...] += jnp.dot(scat, y0, preferred_element_type=jnp.float32)


def _group_call(blkm, xs, w1, b1, w2, b2, srctok3, srcw3):
    grid_spec = pltpu.PrefetchScalarGridSpec(
        num_scalar_prefetch=1,
        grid=(NBLK,),
        in_specs=[
            pl.BlockSpec((BLK, DIM), lambda b, m: (jnp.where(m[b] < E, b, 0), 0)),
            pl.BlockSpec((DIM, HID), lambda b, m: (0, 0)),
            pl.BlockSpec((HID,), lambda b, m: (0,)),
            pl.BlockSpec((1, HID, DIM), lambda b, m: (jnp.minimum(m[b], E - 1), 0, 0)),
            pl.BlockSpec((E, DIM), lambda b, m: (0, 0)),
            pl.BlockSpec((1, 1, BLK), lambda b, m: (b, 0, 0)),
            pl.BlockSpec((1, 1, BLK), lambda b, m: (b, 0, 0)),
        ],
        out_specs=pl.BlockSpec((N, DIM), lambda b, m: (0, 0)),
    )
    return pl.pallas_call(
        _group_body,
        grid_spec=grid_spec,
        out_shape=jax.ShapeDtypeStruct((N, DIM), jnp.float32),
        compiler_params=pltpu.CompilerParams(
            dimension_semantics=("arbitrary",),
        ),
    )(blkm, xs, w1, b1, w2, b2, srctok3, srcw3)


@jax.jit
def kernel(x, gw1, gb1, gw2, gb2, w1, b1, w2, b2):
    i1, i2, wa, wb = _gate_call(x, gw1, gb1, gw2, gb2)
    srctok, srcw, blkm, xs = _route_call(i1, i2, wa, wb, x)
    srctok3 = srctok.reshape(NBLK, 1, BLK)
    srcw3 = srcw.reshape(NBLK, 1, BLK)
    out = _group_call(blkm, xs, w1, b1, w2, b2, srctok3, srcw3)
    return out


# speedup vs baseline: 1.0825x; 1.0825x over previous
import functools

import jax
import jax.numpy as jnp
from jax import lax
from jax.experimental import pallas as pl
from jax.experimental.pallas import tpu as pltpu
from jax.experimental.pallas import tpu_sc as plsc

DIM = 768
HID = 4 * DIM
GDIM = 2 * DIM
E = 8
K = 2
N = 2048
NE = N * K
BLK = 256
NBLK = NE // BLK + E
NSLOT = NBLK * BLK
L = 16
HALF = N // 2

_NEG = -3.0e38


def _silu(v):
    return v * lax.logistic(v)


def _gate_body(x_ref, gw1_ref, gb1_ref, gw2_ref, gb2_ref,
               i1_ref, i2_ref, wa_ref, wb_ref):
    xt = x_ref[...]
    hg = _silu(jnp.dot(xt, gw1_ref[...],
                       preferred_element_type=jnp.float32) + gb1_ref[...])
    g = jnp.dot(hg, gw2_ref[...],
                preferred_element_type=jnp.float32) + gb2_ref[...]
    e_idx = lax.broadcasted_iota(jnp.int32, (N, E), 1)
    max1 = jnp.max(g, axis=1, keepdims=True)
    i1 = jnp.min(jnp.where(g == max1, e_idx, E), axis=1, keepdims=True)
    g2 = jnp.where(e_idx == i1, _NEG, g)
    max2 = jnp.max(g2, axis=1, keepdims=True)
    i2 = jnp.min(jnp.where(g2 == max2, e_idx, E), axis=1, keepdims=True)
    p = jnp.exp(max2 - max1)
    wa = 1.0 / (1.0 + p)
    i1_ref[...] = jnp.min(jnp.where(g == max1, e_idx, E), axis=1)
    i2_ref[...] = jnp.min(jnp.where(g2 == max2, e_idx, E), axis=1)
    wa_ref[...] = wa[:, 0]
    wb_ref[...] = 1.0 - wa[:, 0]


def _gate_call(x, gw1, gb1, gw2, gb2):
    return pl.pallas_call(
        _gate_body,
        out_shape=(
            jax.ShapeDtypeStruct((N,), jnp.int32),
            jax.ShapeDtypeStruct((N,), jnp.int32),
            jax.ShapeDtypeStruct((N,), jnp.float32),
            jax.ShapeDtypeStruct((N,), jnp.float32),
        ),
    )(x, gw1, gb1, gw2, gb2)


def _route_body(i1_hbm, i2_hbm, wa_hbm, wb_hbm, x_hbm,
                srctok_hbm, srcw_hbm, blkm_hbm, xs_hbm,
                eids_v, ws_v, tokb_v, wbuf_v, rows_a, rows_b, blk_v,
                sem_a, sem_b):
    c = lax.axis_index("c")
    s = lax.axis_index("s")
    e_me = s * 2 + c

    @pl.when(s < 4)
    def _active():
        lane = lax.iota(jnp.int32, 16)
        pltpu.sync_copy(i1_hbm, eids_v.at[pl.ds(0, N)])
        pltpu.sync_copy(i2_hbm, eids_v.at[pl.ds(N, N)])
        pltpu.sync_copy(wa_hbm, ws_v.at[pl.ds(0, N)])
        pltpu.sync_copy(wb_hbm, ws_v.at[pl.ds(N, N)])

        def cbody(i, cnts):
            v = eids_v[pl.ds(i * 16, 16)]
            return tuple(
                cnts[e2] + jnp.where(v == e2, 1, 0)
                for e2 in range(E))
        counts_v = lax.fori_loop(
            0, NE // 16, cbody,
            tuple(jnp.zeros((16,), jnp.int32) for _ in range(E)))
        counts = [jnp.sum(counts_v[e2]) for e2 in range(E)]

        nblk = [lax.shift_right_logical(counts[e2] + (BLK - 1), 8)
                for e2 in range(E)]
        base_s = jnp.int32(0)
        nblk_s = jnp.int32(0)
        for e2 in range(E):
            base_s = base_s + jnp.where(e2 < e_me, nblk[e2] * BLK, 0)
            nblk_s = nblk_s + jnp.where(e2 == e_me, nblk[e2], 0)

        zero_i = jnp.zeros((16,), jnp.int32)
        zero_f = jnp.zeros((16,), jnp.float32)

        @pl.when(nblk_s > 0)
        def _ztail():
            zb = pl.multiple_of(base_s + (nblk_s - 1) * BLK, BLK)

            def zbody(i, _):
                tokb_v[pl.ds(zb + i * 16, 16)] = zero_i
                wbuf_v[pl.ds(zb + i * 16, 16)] = zero_f
                return 0
            lax.fori_loop(0, BLK // 16, zbody, 0)

        def pbody(i, off):
            v = eids_v[pl.ds(i * 16, 16)]
            m = v == e_me
            p = i * 16 + lane
            tok = jnp.bitwise_and(p, N - 1)
            wv = ws_v[pl.ds(i * 16, 16)]
            cm = plsc.cumsum(jnp.where(m, 1, 0))
            slot = base_s + off + cm - 1
            plsc.store_scatter(tokb_v, [slot], tok, mask=m)
            plsc.store_scatter(wbuf_v, [slot], wv, mask=m)
            return off + jnp.max(cm)
        lax.fori_loop(0, NE // 16, pbody, jnp.int32(0))

        def dbody(j, _):
            sb = pl.multiple_of(base_s + j * BLK, BLK)
            pltpu.sync_copy(tokb_v.at[pl.ds(sb, BLK)],
                            srctok_hbm.at[pl.ds(sb, BLK)])
            pltpu.sync_copy(wbuf_v.at[pl.ds(sb, BLK)],
                            srcw_hbm.at[pl.ds(sb, BLK)])
            return 0
        lax.fori_loop(0, nblk_s, dbody, 0)

        nsub = nblk_s * (BLK // 64)
        gbase = base_s // 64

        def _fire(k, buf, sem):
            @pl.when(k < nsub)
            def _():
                so = pl.multiple_of((gbase + k) * 64, 64)
                pltpu.async_copy(x_hbm.at[tokb_v.at[pl.ds(so, 64)]],
                                 buf, sem)

        def _drain(k, buf, sem):
            @pl.when(k < nsub)
            def _():
                so = pl.multiple_of((gbase + k) * 64, 64)
                pltpu.make_async_copy(
                    x_hbm.at[tokb_v.at[pl.ds(so, 64)]], buf, sem).wait()
                pltpu.sync_copy(buf, xs_hbm.at[pl.ds(so, 64)])

        _fire(jnp.int32(0), rows_a, sem_a)

        def gbody(j, _):
            k0 = j * 2
            _fire(k0 + 1, rows_b, sem_b)
            _drain(k0, rows_a, sem_a)
            _fire(k0 + 2, rows_a, sem_a)
            _drain(k0 + 1, rows_b, sem_b)
            return 0
        lax.fori_loop(0, (nsub + 1) // 2, gbody, 0)

        @pl.when((c == 0) & (s == 0))
        def _meta():
            cum = jnp.int32(0)
            cums = []
            for e2 in range(E):
                cum = cum + nblk[e2]
                cums.append(cum)
            for step in range(2):
                b = step * 16 + lane
                eid = jnp.zeros((16,), jnp.int32)
                for e2 in range(E):
                    eid = eid + jnp.where(b >= cums[e2], 1, 0)
                blk_v[pl.ds(step * 16, 16)] = eid
            pltpu.sync_copy(blk_v, blkm_hbm)


def _route_call(i1, i2, wa, wb, x):
    mesh = plsc.VectorSubcoreMesh(core_axis_name="c", subcore_axis_name="s",
                                  num_cores=2, num_subcores=16)
    f = pl.kernel(
        _route_body,
        out_type=(
            jax.ShapeDtypeStruct((NSLOT,), jnp.int32),
            jax.ShapeDtypeStruct((NSLOT,), jnp.float32),
            jax.ShapeDtypeStruct((32,), jnp.int32),
            jax.ShapeDtypeStruct((NSLOT, DIM), jnp.float32),
        ),
        mesh=mesh,
        scratch_types=[
            pltpu.VMEM((NE,), jnp.int32),
            pltpu.VMEM((NE,), jnp.float32),
            pltpu.VMEM((NSLOT,), jnp.int32),
            pltpu.VMEM((NSLOT,), jnp.float32),
            pltpu.VMEM((64, DIM), jnp.float32),
            pltpu.VMEM((64, DIM), jnp.float32),
            pltpu.VMEM((32,), jnp.int32),
            pltpu.SemaphoreType.DMA,
            pltpu.SemaphoreType.DMA,
        ],
        compiler_params=pltpu.CompilerParams(needs_layout_passes=False),
    )
    return f(i1, i2, wa, wb, x)


def _group_body(meta_ref, xs_ref, w1_ref, b1_ref, w2_ref, b2_ref, tok_ref,
                w_ref, out_ref):
    b = pl.program_id(0)
    e = meta_ref[b]

    @pl.when(b == 0)
    def _init():
        out_ref[...] = jnp.zeros_like(out_ref)

    @pl.when(e < E)
    def _compute():
        xt = xs_ref[...]
        h = _silu(jnp.dot(xt, w1_ref[...],
                          preferred_element_type=jnp.float32) + b1_ref[...])
        y0 = jnp.dot(h, w2_ref[0], preferred_element_type=jnp.float32)
        onehot = (lax.broadcasted_iota(jnp.int32, (1, E), 1) == e
                  ).astype(jnp.float32)
        y0 = y0 + jnp.dot(onehot, b2_ref[...],
                          preferred_element_type=jnp.float32)
        tok_row = tok_ref[...].reshape(1, BLK)
        w_row = w_ref[...].reshape(1, BLK)
        n_i = lax.broadcasted_iota(jnp.int32, (N, BLK), 0)
        scat = jnp.where(n_i == tok_row, jnp.broadcast_to(w_row, (N, BLK)),
                         0.0)
        out_ref[...] += jnp.dot(scat, y0, preferred_element_type=jnp.float32)


def _group_call(blkm, xs, w1, b1, w2, b2, srctok3, srcw3):
    grid_spec = pltpu.PrefetchScalarGridSpec(
        num_scalar_prefetch=1,
        grid=(NBLK,),
        in_specs=[
            pl.BlockSpec((BLK, DIM), lambda b, m: (jnp.where(m[b] < E, b, 0), 0)),
            pl.BlockSpec((DIM, HID), lambda b, m: (0, 0)),
            pl.BlockSpec((HID,), lambda b, m: (0,)),
            pl.BlockSpec((1, HID, DIM), lambda b, m: (jnp.minimum(m[b], E - 1), 0, 0)),
            pl.BlockSpec((E, DIM), lambda b, m: (0, 0)),
            pl.BlockSpec((1, 1, BLK), lambda b, m: (b, 0, 0)),
            pl.BlockSpec((1, 1, BLK), lambda b, m: (b, 0, 0)),
        ],
        out_specs=pl.BlockSpec((N, DIM), lambda b, m: (0, 0)),
    )
    return pl.pallas_call(
        _group_body,
        grid_spec=grid_spec,
        out_shape=jax.ShapeDtypeStruct((N, DIM), jnp.float32),
        compiler_params=pltpu.CompilerParams(
            dimension_semantics=("arbitrary",),
        ),
    )(blkm, xs, w1, b1, w2, b2, srctok3, srcw3)


@jax.jit
def kernel(x, gw1, gb1, gw2, gb2, w1, b1, w2, b2):
    i1, i2, wa, wb = _gate_call(x, gw1, gb1, gw2, gb2)
    srctok, srcw, blkm, xs = _route_call(i1, i2, wa, wb, x)
    srctok3 = srctok.reshape(NBLK, 1, BLK)
    srcw3 = srcw.reshape(NBLK, 1, BLK)
    out = _group_call(blkm, xs, w1, b1, w2, b2, srctok3, srcw3)
    return out


# speedup vs baseline: 1.1169x; 1.0317x over previous
import functools

import jax
import jax.numpy as jnp
from jax import lax
from jax.experimental import pallas as pl
from jax.experimental.pallas import tpu as pltpu
from jax.experimental.pallas import tpu_sc as plsc

DIM = 768
HID = 4 * DIM
GDIM = 2 * DIM
E = 8
K = 2
N = 2048
NE = N * K
BLK = 256
NBLK = NE // BLK + E
NSLOT = NBLK * BLK
L = 16
HALF = N // 2

_NEG = -3.0e38


def _silu(v):
    return v * lax.logistic(v)


def _gate_body(x_ref, gw1_ref, gb1_ref, gw2_ref, gb2_ref,
               i1_ref, i2_ref, wa_ref, wb_ref):
    xT = jnp.transpose(x_ref[...])
    hgT = _silu(lax.dot_general(
        gw1_ref[...], xT, (((0,), (0,)), ((), ())),
        preferred_element_type=jnp.float32) + gb1_ref[...])
    gT = lax.dot_general(
        gw2_ref[...], hgT, (((0,), (0,)), ((), ())),
        preferred_element_type=jnp.float32) + gb2_ref[...]
    best1 = gT[0:1, :]
    best2 = jnp.full((1, N), _NEG, jnp.float32)
    i1 = jnp.zeros((1, N), jnp.int32)
    i2 = jnp.zeros((1, N), jnp.int32)
    for e in range(1, E):
        ge = gT[e:e + 1, :]
        gt1 = ge > best1
        gt2 = ge > best2
        i2 = jnp.where(gt1, i1, jnp.where(gt2, e, i2))
        best2 = jnp.where(gt1, best1, jnp.where(gt2, ge, best2))
        i1 = jnp.where(gt1, e, i1)
        best1 = jnp.where(gt1, ge, best1)
    p = jnp.exp(best2 - best1)
    wa = 1.0 / (1.0 + p)
    i1_ref[...] = i1
    i2_ref[...] = i2
    wa_ref[...] = wa
    wb_ref[...] = 1.0 - wa


def _gate_call(x, gw1, gb1, gw2, gb2):
    res = pl.pallas_call(
        _gate_body,
        out_shape=(
            jax.ShapeDtypeStruct((1, N), jnp.int32),
            jax.ShapeDtypeStruct((1, N), jnp.int32),
            jax.ShapeDtypeStruct((1, N), jnp.float32),
            jax.ShapeDtypeStruct((1, N), jnp.float32),
        ),
    )(x, gw1, gb1.reshape(GDIM, 1), gw2, gb2.reshape(E, 1))
    return tuple(r.reshape(N) for r in res)


def _route_body(i1_hbm, i2_hbm, wa_hbm, wb_hbm, x_hbm,
                srctok_hbm, srcw_hbm, blkm_hbm, xs_hbm,
                eids_v, ws_v, tokb_v, wbuf_v, rows_a, rows_b, blk_v,
                sem_a, sem_b):
    c = lax.axis_index("c")
    s = lax.axis_index("s")
    e_me = s * 2 + c

    @pl.when(s < 4)
    def _active():
        lane = lax.iota(jnp.int32, 16)
        pltpu.sync_copy(i1_hbm, eids_v.at[pl.ds(0, N)])
        pltpu.sync_copy(i2_hbm, eids_v.at[pl.ds(N, N)])
        pltpu.sync_copy(wa_hbm, ws_v.at[pl.ds(0, N)])
        pltpu.sync_copy(wb_hbm, ws_v.at[pl.ds(N, N)])

        def cbody(i, cnts):
            v = eids_v[pl.ds(i * 16, 16)]
            return tuple(
                cnts[e2] + jnp.where(v == e2, 1, 0)
                for e2 in range(E))
        counts_v = lax.fori_loop(
            0, NE // 16, cbody,
            tuple(jnp.zeros((16,), jnp.int32) for _ in range(E)))
        counts = [jnp.sum(counts_v[e2]) for e2 in range(E)]

        nblk = [lax.shift_right_logical(counts[e2] + (BLK - 1), 8)
                for e2 in range(E)]
        base_s = jnp.int32(0)
        nblk_s = jnp.int32(0)
        for e2 in range(E):
            base_s = base_s + jnp.where(e2 < e_me, nblk[e2] * BLK, 0)
            nblk_s = nblk_s + jnp.where(e2 == e_me, nblk[e2], 0)

        zero_i = jnp.zeros((16,), jnp.int32)
        zero_f = jnp.zeros((16,), jnp.float32)

        @pl.when(nblk_s > 0)
        def _ztail():
            zb = pl.multiple_of(base_s + (nblk_s - 1) * BLK, BLK)

            def zbody(i, _):
                tokb_v[pl.ds(zb + i * 16, 16)] = zero_i
                wbuf_v[pl.ds(zb + i * 16, 16)] = zero_f
                return 0
            lax.fori_loop(0, BLK // 16, zbody, 0)

        def pbody(i, off):
            v = eids_v[pl.ds(i * 16, 16)]
            m = v == e_me
            p = i * 16 + lane
            tok = jnp.bitwise_and(p, N - 1)
            wv = ws_v[pl.ds(i * 16, 16)]
            cm = plsc.cumsum(jnp.where(m, 1, 0))
            slot = base_s + off + cm - 1
            plsc.store_scatter(tokb_v, [slot], tok, mask=m)
            plsc.store_scatter(wbuf_v, [slot], wv, mask=m)
            return off + jnp.max(cm)
        lax.fori_loop(0, NE // 16, pbody, jnp.int32(0))

        def dbody(j, _):
            sb = pl.multiple_of(base_s + j * BLK, BLK)
            pltpu.sync_copy(tokb_v.at[pl.ds(sb, BLK)],
                            srctok_hbm.at[pl.ds(sb, BLK)])
            pltpu.sync_copy(wbuf_v.at[pl.ds(sb, BLK)],
                            srcw_hbm.at[pl.ds(sb, BLK)])
            return 0
        lax.fori_loop(0, nblk_s, dbody, 0)

        nsub = nblk_s * (BLK // 64)
        gbase = base_s // 64

        def _fire(k, buf, sem):
            @pl.when(k < nsub)
            def _():
                so = pl.multiple_of((gbase + k) * 64, 64)
                pltpu.async_copy(x_hbm.at[tokb_v.at[pl.ds(so, 64)]],
                                 buf, sem)

        def _drain(k, buf, sem):
            @pl.when(k < nsub)
            def _():
                so = pl.multiple_of((gbase + k) * 64, 64)
                pltpu.make_async_copy(
                    x_hbm.at[tokb_v.at[pl.ds(so, 64)]], buf, sem).wait()
                pltpu.sync_copy(buf, xs_hbm.at[pl.ds(so, 64)])

        _fire(jnp.int32(0), rows_a, sem_a)

        def gbody(j, _):
            k0 = j * 2
            _fire(k0 + 1, rows_b, sem_b)
            _drain(k0, rows_a, sem_a)
            _fire(k0 + 2, rows_a, sem_a)
            _drain(k0 + 1, rows_b, sem_b)
            return 0
        lax.fori_loop(0, (nsub + 1) // 2, gbody, 0)

        @pl.when((c == 0) & (s == 0))
        def _meta():
            cum = jnp.int32(0)
            cums = []
            for e2 in range(E):
                cum = cum + nblk[e2]
                cums.append(cum)
            for step in range(2):
                b = step * 16 + lane
                eid = jnp.zeros((16,), jnp.int32)
                for e2 in range(E):
                    eid = eid + jnp.where(b >= cums[e2], 1, 0)
                blk_v[pl.ds(step * 16, 16)] = eid
            pltpu.sync_copy(blk_v, blkm_hbm)


def _route_call(i1, i2, wa, wb, x):
    mesh = plsc.VectorSubcoreMesh(core_axis_name="c", subcore_axis_name="s",
                                  num_cores=2, num_subcores=16)
    f = pl.kernel(
        _route_body,
        out_type=(
            jax.ShapeDtypeStruct((NSLOT,), jnp.int32),
            jax.ShapeDtypeStruct((NSLOT,), jnp.float32),
            jax.ShapeDtypeStruct((32,), jnp.int32),
            jax.ShapeDtypeStruct((NSLOT, DIM), jnp.float32),
        ),
        mesh=mesh,
        scratch_types=[
            pltpu.VMEM((NE,), jnp.int32),
            pltpu.VMEM((NE,), jnp.float32),
            pltpu.VMEM((NSLOT,), jnp.int32),
            pltpu.VMEM((NSLOT,), jnp.float32),
            pltpu.VMEM((64, DIM), jnp.float32),
            pltpu.VMEM((64, DIM), jnp.float32),
            pltpu.VMEM((32,), jnp.int32),
            pltpu.SemaphoreType.DMA,
            pltpu.SemaphoreType.DMA,
        ],
        compiler_params=pltpu.CompilerParams(needs_layout_passes=False),
    )
    return f(i1, i2, wa, wb, x)


def _group_body(meta_ref, xs_ref, w1_ref, b1_ref, w2_ref, b2_ref, tok_ref,
                w_ref, out_ref):
    b = pl.program_id(0)
    e = meta_ref[b]

    @pl.when(b == 0)
    def _init():
        out_ref[...] = jnp.zeros_like(out_ref)

    @pl.when(e < E)
    def _compute():
        xt = xs_ref[...]
        h = _silu(jnp.dot(xt, w1_ref[...],
                          preferred_element_type=jnp.float32) + b1_ref[...])
        y0 = jnp.dot(h, w2_ref[0], preferred_element_type=jnp.float32)
        onehot = (lax.broadcasted_iota(jnp.int32, (1, E), 1) == e
                  ).astype(jnp.float32)
        y0 = y0 + jnp.dot(onehot, b2_ref[...],
                          preferred_element_type=jnp.float32)
        tok_row = tok_ref[...].reshape(1, BLK)
        w_row = w_ref[...].reshape(1, BLK)
        n_i = lax.broadcasted_iota(jnp.int32, (N, BLK), 0)
        scat = jnp.where(n_i == tok_row, jnp.broadcast_to(w_row, (N, BLK)),
                         0.0)
        out_ref[...] += jnp.dot(scat, y0, preferred_element_type=jnp.float32)


def _group_call(blkm, xs, w1, b1, w2, b2, srctok3, srcw3):
    grid_spec = pltpu.PrefetchScalarGridSpec(
        num_scalar_prefetch=1,
        grid=(NBLK,),
        in_specs=[
            pl.BlockSpec((BLK, DIM), lambda b, m: (jnp.where(m[b] < E, b, 0), 0)),
            pl.BlockSpec((DIM, HID), lambda b, m: (0, 0)),
            pl.BlockSpec((HID,), lambda b, m: (0,)),
            pl.BlockSpec((1, HID, DIM), lambda b, m: (jnp.minimum(m[b], E - 1), 0, 0)),
            pl.BlockSpec((E, DIM), lambda b, m: (0, 0)),
            pl.BlockSpec((1, 1, BLK), lambda b, m: (b, 0, 0)),
            pl.BlockSpec((1, 1, BLK), lambda b, m: (b, 0, 0)),
        ],
        out_specs=pl.BlockSpec((N, DIM), lambda b, m: (0, 0)),
    )
    return pl.pallas_call(
        _group_body,
        grid_spec=grid_spec,
        out_shape=jax.ShapeDtypeStruct((N, DIM), jnp.float32),
        compiler_params=pltpu.CompilerParams(
            dimension_semantics=("arbitrary",),
        ),
    )(blkm, xs, w1, b1, w2, b2, srctok3, srcw3)


@jax.jit
def kernel(x, gw1, gb1, gw2, gb2, w1, b1, w2, b2):
    i1, i2, wa, wb = _gate_call(x, gw1, gb1, gw2, gb2)
    srctok, srcw, blkm, xs = _route_call(i1, i2, wa, wb, x)
    srctok3 = srctok.reshape(NBLK, 1, BLK)
    srcw3 = srcw.reshape(NBLK, 1, BLK)
    out = _group_call(blkm, xs, w1, b1, w2, b2, srctok3, srcw3)
    return out


# speedup vs baseline: 1.1416x; 1.0222x over previous
import functools

import jax
import jax.numpy as jnp
from jax import lax
from jax.experimental import pallas as pl
from jax.experimental.pallas import tpu as pltpu
from jax.experimental.pallas import tpu_sc as plsc

DIM = 768
HID = 4 * DIM
GDIM = 2 * DIM
E = 8
K = 2
N = 2048
NE = N * K
BLK = 256
NBLK = NE // BLK + E
NSLOT = NBLK * BLK
L = 16
HALF = N // 2

_NEG = -3.0e38


def _silu(v):
    return v * lax.logistic(v)


def _gate_body(x_ref, gw1_ref, gb1_ref, gw2_ref, gb2_ref,
               i1_ref, i2_ref, wa_ref, wb_ref):
    xT = jnp.transpose(x_ref[...])
    hgT = _silu(lax.dot_general(
        gw1_ref[...], xT, (((0,), (0,)), ((), ())),
        preferred_element_type=jnp.float32) + gb1_ref[...])
    gT = lax.dot_general(
        gw2_ref[...], hgT, (((0,), (0,)), ((), ())),
        preferred_element_type=jnp.float32) + gb2_ref[...]
    best1 = gT[0:1, :]
    best2 = jnp.full((1, N), _NEG, jnp.float32)
    i1 = jnp.zeros((1, N), jnp.int32)
    i2 = jnp.zeros((1, N), jnp.int32)
    for e in range(1, E):
        ge = gT[e:e + 1, :]
        gt1 = ge > best1
        gt2 = ge > best2
        i2 = jnp.where(gt1, i1, jnp.where(gt2, e, i2))
        best2 = jnp.where(gt1, best1, jnp.where(gt2, ge, best2))
        i1 = jnp.where(gt1, e, i1)
        best1 = jnp.where(gt1, ge, best1)
    p = jnp.exp(best2 - best1)
    wa = 1.0 / (1.0 + p)
    i1_ref[...] = i1
    i2_ref[...] = i2
    wa_ref[...] = wa
    wb_ref[...] = 1.0 - wa


def _gate_call(x, gw1, gb1, gw2, gb2):
    res = pl.pallas_call(
        _gate_body,
        out_shape=(
            jax.ShapeDtypeStruct((1, N), jnp.int32),
            jax.ShapeDtypeStruct((1, N), jnp.int32),
            jax.ShapeDtypeStruct((1, N), jnp.float32),
            jax.ShapeDtypeStruct((1, N), jnp.float32),
        ),
    )(x, gw1, gb1.reshape(GDIM, 1), gw2, gb2.reshape(E, 1))
    return tuple(r.reshape(N) for r in res)


def _route_body(i1_hbm, i2_hbm, wa_hbm, wb_hbm, x_hbm,
                srctok_hbm, srcw_hbm, blkm_hbm, xs_hbm,
                eids_v, ws_v, tokb_v, wbuf_v, rows_a, rows_b, blk_v,
                idx_a, idx_b, sem_a, sem_b):
    c = lax.axis_index("c")
    s = lax.axis_index("s")
    lane = lax.iota(jnp.int32, 16)
    is_w = s < 4
    is_h = jnp.logical_and(s >= 8, s < 12)
    e_me = jnp.where(s < 8, s, s - 8) * 2 + c

    pltpu.sync_copy(i1_hbm, eids_v.at[pl.ds(0, N)])
    pltpu.sync_copy(i2_hbm, eids_v.at[pl.ds(N, N)])
    pltpu.sync_copy(wa_hbm, ws_v.at[pl.ds(0, N)])
    pltpu.sync_copy(wb_hbm, ws_v.at[pl.ds(N, N)])

    def cbody(i, cnts):
        v = eids_v[pl.ds(i * 16, 16)]
        return tuple(
            cnts[e2] + jnp.where(v == e2, 1, 0)
            for e2 in range(E))
    counts_v = lax.fori_loop(
        0, NE // 16, cbody,
        tuple(jnp.zeros((16,), jnp.int32) for _ in range(E)))
    counts = [jnp.sum(counts_v[e2]) for e2 in range(E)]

    nblk = [lax.shift_right_logical(counts[e2] + (BLK - 1), 8)
            for e2 in range(E)]
    base_s = jnp.int32(0)
    nblk_s = jnp.int32(0)
    for e2 in range(E):
        base_s = base_s + jnp.where(e2 < e_me, nblk[e2] * BLK, 0)
        nblk_s = nblk_s + jnp.where(e2 == e_me, nblk[e2], 0)
    nsub = nblk_s * (BLK // 64)
    gbase = base_s // 64

    @pl.when(is_w)
    def _route():
        zero_i = jnp.zeros((16,), jnp.int32)
        zero_f = jnp.zeros((16,), jnp.float32)

        @pl.when(nblk_s > 0)
        def _ztail():
            zb = pl.multiple_of(base_s + (nblk_s - 1) * BLK, BLK)

            def zbody(i, _):
                tokb_v[pl.ds(zb + i * 16, 16)] = zero_i
                wbuf_v[pl.ds(zb + i * 16, 16)] = zero_f
                return 0
            lax.fori_loop(0, BLK // 16, zbody, 0)

        def pbody(i, off):
            v = eids_v[pl.ds(i * 16, 16)]
            m = v == e_me
            p = i * 16 + lane
            tok = jnp.bitwise_and(p, N - 1)
            wv = ws_v[pl.ds(i * 16, 16)]
            cm = plsc.cumsum(jnp.where(m, 1, 0))
            slot = base_s + off + cm - 1
            plsc.store_scatter(tokb_v, [slot], tok, mask=m)
            plsc.store_scatter(wbuf_v, [slot], wv, mask=m)
            return off + jnp.max(cm)
        lax.fori_loop(0, NE // 16, pbody, jnp.int32(0))

        def dbody(j, _):
            sb = pl.multiple_of(base_s + j * BLK, BLK)
            pltpu.sync_copy(tokb_v.at[pl.ds(sb, BLK)],
                            srctok_hbm.at[pl.ds(sb, BLK)])
            pltpu.sync_copy(wbuf_v.at[pl.ds(sb, BLK)],
                            srcw_hbm.at[pl.ds(sb, BLK)])
            return 0
        lax.fori_loop(0, nblk_s, dbody, 0)

        @pl.when((c == 0) & (s == 0))
        def _meta():
            cum = jnp.int32(0)
            cums = []
            for e2 in range(E):
                cum = cum + nblk[e2]
                cums.append(cum)
            for step in range(2):
                b = step * 16 + lane
                eid = jnp.zeros((16,), jnp.int32)
                for e2 in range(E):
                    eid = eid + jnp.where(b >= cums[e2], 1, 0)
                blk_v[pl.ds(step * 16, 16)] = eid
            pltpu.sync_copy(blk_v, blkm_hbm)

    plsc.subcore_barrier()

    wh = (nsub + 1) // 2

    @pl.when(is_w)
    def _gather_w():
        def _fire(k, buf, sem):
            @pl.when(k < wh)
            def _():
                so = pl.multiple_of((gbase + k) * 64, 64)
                pltpu.async_copy(x_hbm.at[tokb_v.at[pl.ds(so, 64)]],
                                 buf, sem)

        def _drain(k, buf, sem):
            @pl.when(k < wh)
            def _():
                so = pl.multiple_of((gbase + k) * 64, 64)
                pltpu.make_async_copy(
                    x_hbm.at[tokb_v.at[pl.ds(so, 64)]], buf, sem).wait()
                pltpu.sync_copy(buf, xs_hbm.at[pl.ds(so, 64)])

        _fire(jnp.int32(0), rows_a, sem_a)

        def gbody(j, _):
            k0 = j * 2
            _fire(k0 + 1, rows_b, sem_b)
            _drain(k0, rows_a, sem_a)
            _fire(k0 + 2, rows_a, sem_a)
            _drain(k0 + 1, rows_b, sem_b)
            return 0
        lax.fori_loop(0, (wh + 1) // 2, gbody, 0)

    @pl.when(is_h)
    def _gather_h():
        def _fire(k, ib, buf, sem):
            @pl.when(k < nsub)
            def _():
                so = pl.multiple_of((gbase + k) * 64, 64)
                pltpu.sync_copy(srctok_hbm.at[pl.ds(so, 64)], ib)
                pltpu.async_copy(x_hbm.at[ib], buf, sem)

        def _drain(k, ib, buf, sem):
            @pl.when(k < nsub)
            def _():
                so = pl.multiple_of((gbase + k) * 64, 64)
                pltpu.make_async_copy(x_hbm.at[ib], buf, sem).wait()
                pltpu.sync_copy(buf, xs_hbm.at[pl.ds(so, 64)])

        _fire(wh, idx_a, rows_a, sem_a)

        def gbody(j, _):
            k0 = wh + j * 2
            _fire(k0 + 1, idx_b, rows_b, sem_b)
            _drain(k0, idx_a, rows_a, sem_a)
            _fire(k0 + 2, idx_a, rows_a, sem_a)
            _drain(k0 + 1, idx_b, rows_b, sem_b)
            return 0
        lax.fori_loop(0, (nsub - wh + 1) // 2, gbody, 0)


def _route_call(i1, i2, wa, wb, x):
    mesh = plsc.VectorSubcoreMesh(core_axis_name="c", subcore_axis_name="s",
                                  num_cores=2, num_subcores=16)
    f = pl.kernel(
        _route_body,
        out_type=(
            jax.ShapeDtypeStruct((NSLOT,), jnp.int32),
            jax.ShapeDtypeStruct((NSLOT,), jnp.float32),
            jax.ShapeDtypeStruct((32,), jnp.int32),
            jax.ShapeDtypeStruct((NSLOT, DIM), jnp.float32),
        ),
        mesh=mesh,
        scratch_types=[
            pltpu.VMEM((NE,), jnp.int32),
            pltpu.VMEM((NE,), jnp.float32),
            pltpu.VMEM((NSLOT,), jnp.int32),
            pltpu.VMEM((NSLOT,), jnp.float32),
            pltpu.VMEM((64, DIM), jnp.float32),
            pltpu.VMEM((64, DIM), jnp.float32),
            pltpu.VMEM((32,), jnp.int32),
            pltpu.VMEM((64,), jnp.int32),
            pltpu.VMEM((64,), jnp.int32),
            pltpu.SemaphoreType.DMA,
            pltpu.SemaphoreType.DMA,
        ],
        compiler_params=pltpu.CompilerParams(needs_layout_passes=False),
    )
    return f(i1, i2, wa, wb, x)


def _group_body(meta_ref, xs_ref, w1_ref, b1_ref, w2_ref, b2_ref, tok_ref,
                w_ref, out_ref):
    b = pl.program_id(0)
    e = meta_ref[b]

    @pl.when(b == 0)
    def _init():
        out_ref[...] = jnp.zeros_like(out_ref)

    @pl.when(e < E)
    def _compute():
        xt = xs_ref[...]
        h = _silu(jnp.dot(xt, w1_ref[...],
                          preferred_element_type=jnp.float32) + b1_ref[...])
        y0 = jnp.dot(h, w2_ref[0], preferred_element_type=jnp.float32)
        onehot = (lax.broadcasted_iota(jnp.int32, (1, E), 1) == e
                  ).astype(jnp.float32)
        y0 = y0 + jnp.dot(onehot, b2_ref[...],
                          preferred_element_type=jnp.float32)
        tok_row = tok_ref[...].reshape(1, BLK)
        w_row = w_ref[...].reshape(1, BLK)
        n_i = lax.broadcasted_iota(jnp.int32, (N, BLK), 0)
        scat = jnp.where(n_i == tok_row, jnp.broadcast_to(w_row, (N, BLK)),
                         0.0)
        out_ref[...] += jnp.dot(scat, y0, preferred_element_type=jnp.float32)


def _group_call(blkm, xs, w1, b1, w2, b2, srctok3, srcw3):
    grid_spec = pltpu.PrefetchScalarGridSpec(
        num_scalar_prefetch=1,
        grid=(NBLK,),
        in_specs=[
            pl.BlockSpec((BLK, DIM), lambda b, m: (jnp.where(m[b] < E, b, 0), 0)),
            pl.BlockSpec((DIM, HID), lambda b, m: (0, 0)),
            pl.BlockSpec((HID,), lambda b, m: (0,)),
            pl.BlockSpec((1, HID, DIM), lambda b, m: (jnp.minimum(m[b], E - 1), 0, 0)),
            pl.BlockSpec((E, DIM), lambda b, m: (0, 0)),
            pl.BlockSpec((1, 1, BLK), lambda b, m: (b, 0, 0)),
            pl.BlockSpec((1, 1, BLK), lambda b, m: (b, 0, 0)),
        ],
        out_specs=pl.BlockSpec((N, DIM), lambda b, m: (0, 0)),
    )
    return pl.pallas_call(
        _group_body,
        grid_spec=grid_spec,
        out_shape=jax.ShapeDtypeStruct((N, DIM), jnp.float32),
        compiler_params=pltpu.CompilerParams(
            dimension_semantics=("arbitrary",),
        ),
    )(blkm, xs, w1, b1, w2, b2, srctok3, srcw3)


@jax.jit
def kernel(x, gw1, gb1, gw2, gb2, w1, b1, w2, b2):
    i1, i2, wa, wb = _gate_call(x, gw1, gb1, gw2, gb2)
    srctok, srcw, blkm, xs = _route_call(i1, i2, wa, wb, x)
    srctok3 = srctok.reshape(NBLK, 1, BLK)
    srcw3 = srcw.reshape(NBLK, 1, BLK)
    out = _group_call(blkm, xs, w1, b1, w2, b2, srctok3, srcw3)
    return out
